# baseline (speedup 1.0000x reference)
import jax
import jax.numpy as jnp
from jax import lax
from jax.experimental import pallas as pl
from jax.experimental.pallas import tpu as pltpu
from jax.experimental.pallas import tpu_sc as plsc

BATCH = 16384
NROWS = 1000000
D0, D1, D2 = 16, 32, 64

_CC = 2048
_CGRID = -(-NROWS // _CC)


def _make_repack(d):
    rpp = 128 // d

    def body(in_ref, out_ref):
        z = jnp.transpose(in_ref[...])
        z3 = jnp.reshape(z, (_CC // rpp, rpp, d))
        for a in range(rpp):
            out_ref[:, pl.ds(a * d, d)] = z3[:, a, :]

    return pl.pallas_call(
        body,
        grid=(_CGRID,),
        in_specs=[pl.BlockSpec((d, _CC), lambda c: (0, c))],
        out_specs=pl.BlockSpec((_CC * d // 128, 128), lambda c: (c, 0)),
        out_shape=jax.ShapeDtypeStruct(
            (_CGRID * _CC * d // 128, 128), jnp.float32),
    )


_repack0 = _make_repack(D0)
_repack1 = _make_repack(D1)
_repack2 = _make_repack(D2)

_NC = 2
_NS = 16
_NW = _NC * _NS
_BPW = BATCH // _NW
_CHUNK = 128
_NCHUNK = _BPW // _CHUNK

_TABLES = (
    (D0, 3),
    (D1, 2),
    (D2, 1),
)


def _sc_body(idx_hbm, w0, w1, w2, o0, o1, o2,
             idx_v, g0, g1, g2, f0, f1, f2, gbuf, obuf,
             sa, sb, soa, sob):
    wid = lax.axis_index("s") * _NC + lax.axis_index("c")
    base = wid * _BPW

    pltpu.sync_copy(idx_hbm.at[pl.ds(base, _BPW)], idx_v)

    for gref, offr, (d, sh) in ((g0, f0, _TABLES[0]), (g1, f1, _TABLES[1]),
                                (g2, f2, _TABLES[2])):
        mask = (1 << sh) - 1
        dlog = d.bit_length() - 1
        for s in range(_BPW // 16):
            v = idx_v[pl.ds(s * 16, 16)]
            dst = (s // 8, pl.ds((s % 8) * 16, 16))
            gref[dst[0], dst[1]] = lax.shift_right_logical(v, sh)
            offr[dst[0], dst[1]] = lax.shift_left(v & mask, dlog)

    work = []
    for (w, gref, offr, o, (d, _)) in (
            (w0, g0, f0, o0, _TABLES[0]),
            (w1, g1, f1, o1, _TABLES[1]),
            (w2, g2, f2, o2, _TABLES[2])):
        for j in range(_NCHUNK):
            work.append((w, gref, offr, o, d, j))

    gsems = (sa, sb)
    osems = (soa, sob)
    iota = lax.iota(jnp.int32, 16)

    def fire(item, slot):
        w, gref, _, _, _, j = item
        return pltpu.async_copy(w.at[gref.at[j]], gbuf.at[slot], gsems[slot])

    def extract(item, slot):
        w, gref, offr, o, d, j = item
        nsl = d // 16

        def body(k, carry):
            kk = jnp.full((16,), k, jnp.int32)
            off = plsc.load_gather(offr.at[j], [kk])
            col = off + iota
            for c in range(nsl):
                v = plsc.load_gather(gbuf.at[slot], [kk, col + c * 16])
                obuf[slot, pl.ds(k * d + c * 16, 16)] = jnp.maximum(v, 0.0)
            return carry

        lax.fori_loop(0, _CHUNK, body, 0)
        return pltpu.async_copy(
            obuf.at[slot, pl.ds(0, _CHUNK * d)],
            o.at[pl.ds((base + j * _CHUNK) * d, _CHUNK * d)], osems[slot])

    copies = [fire(work[0], 0), fire(work[1], 1)]
    outs = [None, None]
    for n, item in enumerate(work):
        slot = n % 2
        copies[n].wait()
        if outs[slot] is not None:
            outs[slot].wait()
        outs[slot] = extract(item, slot)
        if n + 2 < len(work):
            copies.append(fire(work[n + 2], slot))
    outs[0].wait()
    outs[1].wait()


_gather_relu = pl.kernel(
    _sc_body,
    out_type=(
        jax.ShapeDtypeStruct((BATCH * D0,), jnp.float32),
        jax.ShapeDtypeStruct((BATCH * D1,), jnp.float32),
        jax.ShapeDtypeStruct((BATCH * D2,), jnp.float32),
    ),
    mesh=plsc.VectorSubcoreMesh(core_axis_name="c", subcore_axis_name="s"),
    compiler_params=pltpu.CompilerParams(
        use_tc_tiling_on_sc=True, needs_layout_passes=False),
    scratch_types=[
        pltpu.VMEM((_BPW,), jnp.int32),
        pltpu.VMEM((_NCHUNK, _CHUNK), jnp.int32),
        pltpu.VMEM((_NCHUNK, _CHUNK), jnp.int32),
        pltpu.VMEM((_NCHUNK, _CHUNK), jnp.int32),
        pltpu.VMEM((_NCHUNK, _CHUNK), jnp.int32),
        pltpu.VMEM((_NCHUNK, _CHUNK), jnp.int32),
        pltpu.VMEM((_NCHUNK, _CHUNK), jnp.int32),
        pltpu.VMEM((2, _CHUNK, 128), jnp.float32),
        pltpu.VMEM((2, _CHUNK * D2), jnp.float32),
        pltpu.SemaphoreType.DMA,
        pltpu.SemaphoreType.DMA,
        pltpu.SemaphoreType.DMA,
        pltpu.SemaphoreType.DMA,
    ],
)


def kernel(indices, W0, W1, W2):
    idx = indices.astype(jnp.int32)
    p0 = _repack0(jnp.swapaxes(W0, 0, 1))
    p1 = _repack1(jnp.swapaxes(W1, 0, 1))
    p2 = _repack2(jnp.swapaxes(W2, 0, 1))
    f0, f1, f2 = _gather_relu(idx, p0, p1, p2)
    return (f0.reshape(BATCH, D0), f1.reshape(BATCH, D1),
            f2.reshape(BATCH, D2))


# speedup vs baseline: 1.2602x; 1.2602x over previous
import jax
import jax.numpy as jnp
from jax import lax
from jax.experimental import pallas as pl
from jax.experimental.pallas import tpu as pltpu
from jax.experimental.pallas import tpu_sc as plsc

BATCH = 16384
NROWS = 1000000
D0, D1, D2 = 16, 32, 64
_SEG = ((D0, 0), (D1, D0), (D2, D0 + D1))

_NC = 2
_NS = 16
_NW = _NC * _NS
_BPW = BATCH // _NW
_CHUNK = 128
_NCHUNK = _BPW // _CHUNK
_OSTRIDE = D0 + D1 + D2


def _sc_body(idx_hbm, wcat, o0, o1, o2, idx_v, gbuf, obuf, sa, sb, soa, sob):
    wid = lax.axis_index("s") * _NC + lax.axis_index("c")
    base = wid * _BPW

    pltpu.sync_copy(idx_hbm.at[pl.ds(base, _BPW)], idx_v)

    gsems = (sa, sb)
    osems = (soa, sob)

    def fire(j, slot):
        return pltpu.async_copy(
            wcat.at[idx_v.at[pl.ds(j * _CHUNK, _CHUNK)]],
            gbuf.at[slot], gsems[slot])

    def extract(j, slot):
        def body(k, carry):
            pos = 0
            for (d, col) in _SEG:
                for c in range(d // 16):
                    v = gbuf[slot, k, pl.ds(col + c * 16, 16)]
                    obuf[slot, pl.ds(pos * _CHUNK + k * d + c * 16, 16)] = (
                        jnp.maximum(v, 0.0))
                pos += d
            return carry

        lax.fori_loop(0, _CHUNK, body, 0)
        outs = []
        pos = 0
        for (d, _), o in zip(_SEG, (o0, o1, o2)):
            outs.append(pltpu.async_copy(
                obuf.at[slot, pl.ds(pos * _CHUNK, _CHUNK * d)],
                o.at[pl.ds((base + j * _CHUNK) * d, _CHUNK * d)],
                osems[slot]))
            pos += d
        return outs

    copies = [fire(0, 0), fire(1, 1)]
    outs = [None, None]
    for j in range(_NCHUNK):
        slot = j % 2
        copies[j].wait()
        if outs[slot] is not None:
            for c in outs[slot]:
                c.wait()
        outs[slot] = extract(j, slot)
        if j + 2 < _NCHUNK:
            copies.append(fire(j + 2, slot))
    for group in outs:
        for c in group:
            c.wait()


def _obuf_layout_note():
    pass


_gather_relu = pl.kernel(
    _sc_body,
    out_type=(
        jax.ShapeDtypeStruct((BATCH * D0,), jnp.float32),
        jax.ShapeDtypeStruct((BATCH * D1,), jnp.float32),
        jax.ShapeDtypeStruct((BATCH * D2,), jnp.float32),
    ),
    mesh=plsc.VectorSubcoreMesh(core_axis_name="c", subcore_axis_name="s"),
    compiler_params=pltpu.CompilerParams(
        use_tc_tiling_on_sc=True, needs_layout_passes=False),
    scratch_types=[
        pltpu.VMEM((_BPW,), jnp.int32),
        pltpu.VMEM((2, _CHUNK, 128), jnp.float32),
        pltpu.VMEM((2, _CHUNK * _OSTRIDE), jnp.float32),
        pltpu.SemaphoreType.DMA,
        pltpu.SemaphoreType.DMA,
        pltpu.SemaphoreType.DMA,
        pltpu.SemaphoreType.DMA,
    ],
)


def kernel(indices, W0, W1, W2):
    idx = indices.astype(jnp.int32)
    wcat = jnp.pad(jnp.concatenate((W0, W1, W2), axis=1), ((0, 0), (0, 16)))
    f0, f1, f2 = _gather_relu(idx, wcat)
    return (f0.reshape(BATCH, D0), f1.reshape(BATCH, D1),
            f2.reshape(BATCH, D2))
